# Initial kernel scaffold; baseline (speedup 1.0000x reference)
import functools

import jax
import jax.numpy as jnp
from jax import lax
from jax.experimental import pallas as pl
from jax.experimental.pallas import tpu as pltpu
from jax.experimental.pallas import tpu_sc as plsc

N_NODES = 10000
N_EDGES = 160000
D_IN = 288
D_H1 = 16
D_H2 = 32
D_OUT = 7

NC = 2
NS = 16
NW = NC * NS

N_PAD = 10240
ROWS_PER_TILE = N_PAD // NS
PAD_DST = N_NODES + 8
PAD_SRC = N_PAD - 1

CHUNK = 128
E_PER_W = 5120
N_CHUNKS = E_PER_W // CHUNK
E_PAD = NW * E_PER_W


def _sc_mesh():
  return plsc.VectorSubcoreMesh(core_axis_name="c", subcore_axis_name="s")


def _deg_kernel(dst_hbm, ones_hbm, zeros_hbm, out_hbm,
                idx_v, ones_v, acc_sh, sem):
  c = lax.axis_index("c")
  s = lax.axis_index("s")
  row0 = s * ROWS_PER_TILE

  pltpu.sync_copy(zeros_hbm.at[pl.ds(row0, ROWS_PER_TILE)],
                  acc_sh.at[pl.ds(row0, ROWS_PER_TILE)])
  pltpu.sync_copy(ones_hbm, ones_v)
  pltpu.sync_copy(dst_hbm.at[c, s], idx_v)
  plsc.subcore_barrier()

  def body(j, carry):
    pltpu.sync_copy(ones_v, acc_sh.at[idx_v.at[j]], add=True)
    return carry

  lax.fori_loop(0, N_CHUNKS, body, 0)
  plsc.subcore_barrier()
  pltpu.sync_copy(acc_sh.at[pl.ds(row0, ROWS_PER_TILE)],
                  out_hbm.at[c, pl.ds(row0, ROWS_PER_TILE)])
  return ()


def _make_deg_call():
  return pl.kernel(
      _deg_kernel,
      out_type=jax.ShapeDtypeStruct((NC, N_PAD, 16), jnp.float32),
      mesh=_sc_mesh(),
      scratch_types=[
          pltpu.VMEM((N_CHUNKS, CHUNK), jnp.int32),
          pltpu.VMEM((CHUNK, 16), jnp.float32),
          pltpu.VMEM_SHARED((N_PAD, 16), jnp.float32),
          pltpu.SemaphoreType.DMA,
      ],
  )


def _edge_agg_kernel(src_hbm, dst_hbm, table_hbm, zeros_hbm, out_hbm,
                     src_v, dst_v, rows_v, acc_sh, sem):
  c = lax.axis_index("c")
  s = lax.axis_index("s")
  row0 = s * ROWS_PER_TILE

  pltpu.sync_copy(zeros_hbm.at[pl.ds(row0, ROWS_PER_TILE)],
                  acc_sh.at[pl.ds(row0, ROWS_PER_TILE)])
  pltpu.sync_copy(src_hbm.at[c, s], src_v)
  pltpu.sync_copy(dst_hbm.at[c, s], dst_v)
  plsc.subcore_barrier()

  def body(j, carry):
    pltpu.async_copy(table_hbm.at[src_v.at[j]], rows_v, sem).wait()
    pltpu.sync_copy(rows_v, acc_sh.at[dst_v.at[j]], add=True)
    return carry

  lax.fori_loop(0, N_CHUNKS, body, 0)
  plsc.subcore_barrier()
  pltpu.sync_copy(acc_sh.at[pl.ds(row0, ROWS_PER_TILE)],
                  out_hbm.at[c, pl.ds(row0, ROWS_PER_TILE)])
  return ()


def _make_edge_agg_call(d):
  return pl.kernel(
      _edge_agg_kernel,
      out_type=jax.ShapeDtypeStruct((NC, N_PAD, d), jnp.float32),
      mesh=_sc_mesh(),
      scratch_types=[
          pltpu.VMEM((N_CHUNKS, CHUNK), jnp.int32),
          pltpu.VMEM((N_CHUNKS, CHUNK), jnp.int32),
          pltpu.VMEM((CHUNK, d), jnp.float32),
          pltpu.VMEM_SHARED((N_PAD, d), jnp.float32),
          pltpu.SemaphoreType.DMA,
      ],
  )


def _dinv_from_degp(degp_ref):
  deg = degp_ref[0, :, 0] + degp_ref[1, :, 0] + 1.0
  return lax.rsqrt(deg)


def _tc_b_kernel(xp_ref, w1_ref, degp_ref, xw_ref):
  dinv = _dinv_from_degp(degp_ref)
  xw = jnp.dot(xp_ref[...], w1_ref[...], preferred_element_type=jnp.float32)
  xw_ref[...] = xw * dinv[:, None]


def _tc_d_kernel(acc_ref, xw1_ref, degp_ref, b1_ref, w2_ref, xw2_ref):
  dinv = _dinv_from_degp(degp_ref)
  h1 = dinv[:, None] * (acc_ref[0] + acc_ref[1] + xw1_ref[...]) + b1_ref[...]
  h1 = jnp.maximum(h1, 0.0)
  xw2 = jnp.dot(h1, w2_ref[...], preferred_element_type=jnp.float32)
  xw2_ref[...] = xw2 * dinv[:, None]


def _tc_f_kernel(acc_ref, xw2_ref, degp_ref, b2_ref, wfc_ref, bfc_ref, out_ref):
  dinv = _dinv_from_degp(degp_ref)
  h2 = dinv[:, None] * (acc_ref[0] + acc_ref[1] + xw2_ref[...]) + b2_ref[...]
  h2 = jnp.maximum(h2, 0.0)
  out_ref[...] = (
      jnp.dot(h2, wfc_ref[...], preferred_element_type=jnp.float32)
      + bfc_ref[...]
  )


@jax.jit
def kernel(x, edge_index, W1, b1, W2, b2, Wfc, bfc):
  src = edge_index[0].astype(jnp.int32)
  dst = edge_index[1].astype(jnp.int32)
  n_extra = E_PAD - N_EDGES
  src_p = jnp.concatenate(
      [src, jnp.full((n_extra,), PAD_SRC, jnp.int32)]
  ).reshape(NC, NS, N_CHUNKS, CHUNK)
  dst_p = jnp.concatenate(
      [dst, jnp.full((n_extra,), PAD_DST, jnp.int32)]
  ).reshape(NC, NS, N_CHUNKS, CHUNK)
  xp = jnp.concatenate(
      [x, jnp.zeros((N_PAD - N_NODES, D_IN), jnp.float32)], axis=0)

  ones16 = jnp.ones((CHUNK, 16), jnp.float32)
  zeros16 = jnp.zeros((N_PAD, 16), jnp.float32)
  zeros32 = jnp.zeros((N_PAD, 32), jnp.float32)

  degp = _make_deg_call()(dst_p, ones16, zeros16)

  xw1 = pl.pallas_call(
      _tc_b_kernel,
      out_shape=jax.ShapeDtypeStruct((N_PAD, D_H1), jnp.float32),
  )(xp, W1, degp)

  acc1 = _make_edge_agg_call(D_H1)(src_p, dst_p, xw1, zeros16)

  xw2 = pl.pallas_call(
      _tc_d_kernel,
      out_shape=jax.ShapeDtypeStruct((N_PAD, D_H2), jnp.float32),
  )(acc1, xw1, degp, b1, W2)

  acc2 = _make_edge_agg_call(D_H2)(src_p, dst_p, xw2, zeros32)

  outp = pl.pallas_call(
      _tc_f_kernel,
      out_shape=jax.ShapeDtypeStruct((N_PAD, D_OUT), jnp.float32),
  )(acc2, xw2, degp, b2, Wfc, bfc)

  return outp[:N_NODES]


# speedup vs baseline: 15.0773x; 15.0773x over previous
import functools

import jax
import jax.numpy as jnp
from jax import lax
from jax.experimental import pallas as pl
from jax.experimental.pallas import tpu as pltpu
from jax.experimental.pallas import tpu_sc as plsc

N_NODES = 10000
N_EDGES = 160000
D_IN = 288
D_H1 = 16
D_H2 = 32
D_OUT = 7

NC = 2
NS = 16
NW = NC * NS

N_PAD = 10240
ROWS_PER_TILE = N_PAD // NS
PAD_DST = N_NODES + 8
PAD_SRC = N_PAD - 1

CHUNK = 128
E_PER_W = 5120
N_CHUNKS = E_PER_W // CHUNK
E_PAD = NW * E_PER_W


def _sc_mesh():
  return plsc.VectorSubcoreMesh(core_axis_name="c", subcore_axis_name="s")


def _deg_kernel(dst_hbm, ones_hbm, zeros_hbm, out_hbm,
                idx_v, ones_v, acc_sh, sem):
  c = lax.axis_index("c")
  s = lax.axis_index("s")
  row0 = s * ROWS_PER_TILE

  pltpu.sync_copy(zeros_hbm.at[pl.ds(row0, ROWS_PER_TILE)],
                  acc_sh.at[pl.ds(row0, ROWS_PER_TILE)])
  pltpu.sync_copy(ones_hbm, ones_v)
  pltpu.sync_copy(dst_hbm.at[c, s], idx_v)
  plsc.subcore_barrier()

  def body(j, carry):
    pltpu.sync_copy(ones_v, acc_sh.at[idx_v.at[j]], add=True)
    return carry

  lax.fori_loop(0, N_CHUNKS, body, 0)
  plsc.subcore_barrier()
  pltpu.sync_copy(acc_sh.at[pl.ds(row0, ROWS_PER_TILE)],
                  out_hbm.at[c, pl.ds(row0, ROWS_PER_TILE)])


def _make_deg_call():
  return pl.kernel(
      _deg_kernel,
      out_type=jax.ShapeDtypeStruct((NC, N_PAD, 16), jnp.float32),
      mesh=_sc_mesh(),
      compiler_params=pltpu.CompilerParams(use_tc_tiling_on_sc=False),
      scratch_types=[
          pltpu.VMEM((N_CHUNKS, CHUNK), jnp.int32),
          pltpu.VMEM((CHUNK, 16), jnp.float32),
          pltpu.VMEM_SHARED((N_PAD, 16), jnp.float32),
          pltpu.SemaphoreType.DMA,
      ],
  )


def _edge_agg_kernel(src_hbm, dst_hbm, table_hbm, zeros_hbm, out_hbm,
                     src_v, dst_v, rows_v, acc_sh, sem):
  c = lax.axis_index("c")
  s = lax.axis_index("s")
  row0 = s * ROWS_PER_TILE

  pltpu.sync_copy(zeros_hbm.at[pl.ds(row0, ROWS_PER_TILE)],
                  acc_sh.at[pl.ds(row0, ROWS_PER_TILE)])
  pltpu.sync_copy(src_hbm.at[c, s], src_v)
  pltpu.sync_copy(dst_hbm.at[c, s], dst_v)
  plsc.subcore_barrier()

  def body(j, carry):
    pltpu.async_copy(table_hbm.at[src_v.at[j]], rows_v, sem).wait()
    pltpu.sync_copy(rows_v, acc_sh.at[dst_v.at[j]], add=True)
    return carry

  lax.fori_loop(0, N_CHUNKS, body, 0)
  plsc.subcore_barrier()
  pltpu.sync_copy(acc_sh.at[pl.ds(row0, ROWS_PER_TILE)],
                  out_hbm.at[c, pl.ds(row0, ROWS_PER_TILE)])


def _make_edge_agg_call(d):
  return pl.kernel(
      _edge_agg_kernel,
      out_type=jax.ShapeDtypeStruct((NC, N_PAD, d), jnp.float32),
      mesh=_sc_mesh(),
      compiler_params=pltpu.CompilerParams(use_tc_tiling_on_sc=False),
      scratch_types=[
          pltpu.VMEM((N_CHUNKS, CHUNK), jnp.int32),
          pltpu.VMEM((N_CHUNKS, CHUNK), jnp.int32),
          pltpu.VMEM((CHUNK, d), jnp.float32),
          pltpu.VMEM_SHARED((N_PAD, d), jnp.float32),
          pltpu.SemaphoreType.DMA,
      ],
  )


def _dinv_from_degp(degp_ref):
  deg = degp_ref[0, :, 0] + degp_ref[1, :, 0] + 1.0
  return lax.rsqrt(deg)


def _tc_b_kernel(xp_ref, w1_ref, degp_ref, xw_ref):
  dinv = _dinv_from_degp(degp_ref)
  xw = jnp.dot(xp_ref[...], w1_ref[...], preferred_element_type=jnp.float32)
  xw_ref[...] = xw * dinv[:, None]


def _tc_d_kernel(acc_ref, xw1_ref, degp_ref, b1_ref, w2_ref, xw2_ref):
  dinv = _dinv_from_degp(degp_ref)
  h1 = dinv[:, None] * (acc_ref[0] + acc_ref[1] + xw1_ref[...]) + b1_ref[...]
  h1 = jnp.maximum(h1, 0.0)
  xw2 = jnp.dot(h1, w2_ref[...], preferred_element_type=jnp.float32)
  xw2_ref[...] = xw2 * dinv[:, None]


def _tc_f_kernel(acc_ref, xw2_ref, degp_ref, b2_ref, wfc_ref, bfc_ref, out_ref):
  dinv = _dinv_from_degp(degp_ref)
  h2 = dinv[:, None] * (acc_ref[0] + acc_ref[1] + xw2_ref[...]) + b2_ref[...]
  h2 = jnp.maximum(h2, 0.0)
  out_ref[...] = (
      jnp.dot(h2, wfc_ref[...], preferred_element_type=jnp.float32)
      + bfc_ref[...]
  )


@jax.jit
def kernel(x, edge_index, W1, b1, W2, b2, Wfc, bfc):
  src = edge_index[0].astype(jnp.int32)
  dst = edge_index[1].astype(jnp.int32)
  n_extra = E_PAD - N_EDGES
  src_p = jnp.concatenate(
      [src, jnp.full((n_extra,), PAD_SRC, jnp.int32)]
  ).reshape(NC, NS, N_CHUNKS, CHUNK)
  dst_p = jnp.concatenate(
      [dst, jnp.full((n_extra,), PAD_DST, jnp.int32)]
  ).reshape(NC, NS, N_CHUNKS, CHUNK)
  xp = jnp.concatenate(
      [x, jnp.zeros((N_PAD - N_NODES, D_IN), jnp.float32)], axis=0)

  ones16 = jnp.ones((CHUNK, 16), jnp.float32)
  zeros16 = jnp.zeros((N_PAD, 16), jnp.float32)
  zeros32 = jnp.zeros((N_PAD, 32), jnp.float32)

  degp = _make_deg_call()(dst_p, ones16, zeros16)

  xw1 = pl.pallas_call(
      _tc_b_kernel,
      out_shape=jax.ShapeDtypeStruct((N_PAD, D_H1), jnp.float32),
  )(xp, W1, degp)

  acc1 = _make_edge_agg_call(D_H1)(src_p, dst_p, xw1, zeros16)

  xw2 = pl.pallas_call(
      _tc_d_kernel,
      out_shape=jax.ShapeDtypeStruct((N_PAD, D_H2), jnp.float32),
  )(acc1, xw1, degp, b1, W2)

  acc2 = _make_edge_agg_call(D_H2)(src_p, dst_p, xw2, zeros32)

  outp = pl.pallas_call(
      _tc_f_kernel,
      out_shape=jax.ShapeDtypeStruct((N_PAD, D_OUT), jnp.float32),
  )(acc2, xw2, degp, b2, Wfc, bfc)

  return outp[:N_NODES]


# speedup vs baseline: 21.1135x; 1.4003x over previous
import functools

import jax
import jax.numpy as jnp
from jax import lax
from jax.experimental import pallas as pl
from jax.experimental.pallas import tpu as pltpu
from jax.experimental.pallas import tpu_sc as plsc

N_NODES = 10000
N_EDGES = 160000
D_IN = 288
D_H1 = 16
D_H2 = 32
D_OUT = 7

NC = 2
NS = 16
NW = NC * NS

N_PAD = 10240
ROWS_PER_TILE = N_PAD // NS
PAD_DST = N_NODES + 8
PAD_SRC = N_PAD - 1

CHUNK = 128
E_PER_W = 5120
N_CHUNKS = E_PER_W // CHUNK
E_PAD = NW * E_PER_W


def _sc_mesh():
  return plsc.VectorSubcoreMesh(core_axis_name="c", subcore_axis_name="s")


GRP = 8
N_GRP = N_CHUNKS // GRP


def _deg_kernel(dst_hbm, ones_hbm, zeros_hbm, out_hbm,
                idx_v, ones_v, acc_sh, ssem):
  c = lax.axis_index("c")
  s = lax.axis_index("s")
  row0 = s * ROWS_PER_TILE

  pltpu.sync_copy(zeros_hbm.at[pl.ds(row0, ROWS_PER_TILE)],
                  acc_sh.at[pl.ds(row0, ROWS_PER_TILE)])
  pltpu.sync_copy(ones_hbm, ones_v)
  pltpu.sync_copy(dst_hbm.at[c, s], idx_v)
  plsc.subcore_barrier()

  def body(g, carry):
    jj = g * GRP
    hs = [
        pltpu.async_copy(ones_v, acc_sh.at[idx_v.at[jj + b]],
                         ssem.at[b], add=True)
        for b in range(GRP)
    ]
    for h in hs:
      h.wait()
    return carry

  lax.fori_loop(0, N_GRP, body, 0)
  plsc.subcore_barrier()
  pltpu.sync_copy(acc_sh.at[pl.ds(row0, ROWS_PER_TILE)],
                  out_hbm.at[c, pl.ds(row0, ROWS_PER_TILE)])


def _make_deg_call():
  return pl.kernel(
      _deg_kernel,
      out_type=jax.ShapeDtypeStruct((NC, N_PAD, 16), jnp.float32),
      mesh=_sc_mesh(),
      compiler_params=pltpu.CompilerParams(use_tc_tiling_on_sc=False),
      scratch_types=[
          pltpu.VMEM((N_CHUNKS, CHUNK), jnp.int32),
          pltpu.VMEM((CHUNK, 16), jnp.float32),
          pltpu.VMEM_SHARED((N_PAD, 16), jnp.float32),
          pltpu.SemaphoreType.DMA((GRP,)),
      ],
  )


def _edge_agg_kernel(src_hbm, dst_hbm, table_hbm, zeros_hbm, out_hbm,
                     src_v, dst_v, rows_v, acc_sh, gsem, ssem):
  c = lax.axis_index("c")
  s = lax.axis_index("s")
  row0 = s * ROWS_PER_TILE

  pltpu.sync_copy(zeros_hbm.at[pl.ds(row0, ROWS_PER_TILE)],
                  acc_sh.at[pl.ds(row0, ROWS_PER_TILE)])
  pltpu.sync_copy(src_hbm.at[c, s], src_v)
  pltpu.sync_copy(dst_hbm.at[c, s], dst_v)
  plsc.subcore_barrier()

  def body(g, carry):
    jj = g * GRP
    ghs = [
        pltpu.async_copy(table_hbm.at[src_v.at[jj + b]], rows_v.at[b],
                         gsem.at[b])
        for b in range(GRP)
    ]
    shs = []
    for b in range(GRP):
      ghs[b].wait()
      shs.append(
          pltpu.async_copy(rows_v.at[b], acc_sh.at[dst_v.at[jj + b]],
                           ssem.at[b], add=True))
    for h in shs:
      h.wait()
    return carry

  lax.fori_loop(0, N_GRP, body, 0)
  plsc.subcore_barrier()
  pltpu.sync_copy(acc_sh.at[pl.ds(row0, ROWS_PER_TILE)],
                  out_hbm.at[c, pl.ds(row0, ROWS_PER_TILE)])


def _make_edge_agg_call(d):
  return pl.kernel(
      _edge_agg_kernel,
      out_type=jax.ShapeDtypeStruct((NC, N_PAD, d), jnp.float32),
      mesh=_sc_mesh(),
      compiler_params=pltpu.CompilerParams(use_tc_tiling_on_sc=False),
      scratch_types=[
          pltpu.VMEM((N_CHUNKS, CHUNK), jnp.int32),
          pltpu.VMEM((N_CHUNKS, CHUNK), jnp.int32),
          pltpu.VMEM((GRP, CHUNK, d), jnp.float32),
          pltpu.VMEM_SHARED((N_PAD, d), jnp.float32),
          pltpu.SemaphoreType.DMA((GRP,)),
          pltpu.SemaphoreType.DMA((GRP,)),
      ],
  )


def _dinv_from_degp(degp_ref):
  deg = degp_ref[0, :, 0] + degp_ref[1, :, 0] + 1.0
  return lax.rsqrt(deg)


def _tc_b_kernel(x_ref, w1_ref, degp_ref, xw_ref):
  dinv = _dinv_from_degp(degp_ref)
  xw = jnp.dot(x_ref[...], w1_ref[...], preferred_element_type=jnp.float32)
  xw_ref[:N_NODES] = xw * dinv[:N_NODES, None]
  xw_ref[N_NODES:] = jnp.zeros((N_PAD - N_NODES, D_H1), jnp.float32)


def _tc_d_kernel(acc_ref, xw1_ref, degp_ref, b1_ref, w2_ref, xw2_ref):
  dinv = _dinv_from_degp(degp_ref)
  h1 = dinv[:, None] * (acc_ref[0] + acc_ref[1] + xw1_ref[...]) + b1_ref[...]
  h1 = jnp.maximum(h1, 0.0)
  xw2 = jnp.dot(h1, w2_ref[...], preferred_element_type=jnp.float32)
  xw2_ref[...] = xw2 * dinv[:, None]


def _tc_f_kernel(acc_ref, xw2_ref, degp_ref, b2_ref, wfc_ref, bfc_ref, out_ref):
  dinv = _dinv_from_degp(degp_ref)
  h2 = dinv[:, None] * (acc_ref[0] + acc_ref[1] + xw2_ref[...]) + b2_ref[...]
  h2 = jnp.maximum(h2, 0.0)
  out_ref[...] = (
      jnp.dot(h2, wfc_ref[...], preferred_element_type=jnp.float32)
      + bfc_ref[...]
  )


@jax.jit
def kernel(x, edge_index, W1, b1, W2, b2, Wfc, bfc):
  src = edge_index[0].astype(jnp.int32)
  dst = edge_index[1].astype(jnp.int32)
  n_extra = E_PAD - N_EDGES
  src_p = jnp.concatenate(
      [src, jnp.full((n_extra,), PAD_SRC, jnp.int32)]
  ).reshape(NC, NS, N_CHUNKS, CHUNK)
  dst_p = jnp.concatenate(
      [dst, jnp.full((n_extra,), PAD_DST, jnp.int32)]
  ).reshape(NC, NS, N_CHUNKS, CHUNK)
  ones16 = jnp.ones((CHUNK, 16), jnp.float32)
  zeros16 = jnp.zeros((N_PAD, 16), jnp.float32)
  zeros32 = jnp.zeros((N_PAD, 32), jnp.float32)

  degp = _make_deg_call()(dst_p, ones16, zeros16)

  xw1 = pl.pallas_call(
      _tc_b_kernel,
      out_shape=jax.ShapeDtypeStruct((N_PAD, D_H1), jnp.float32),
  )(x, W1, degp)

  acc1 = _make_edge_agg_call(D_H1)(src_p, dst_p, xw1, zeros16)

  xw2 = pl.pallas_call(
      _tc_d_kernel,
      out_shape=jax.ShapeDtypeStruct((N_PAD, D_H2), jnp.float32),
  )(acc1, xw1, degp, b1, W2)

  acc2 = _make_edge_agg_call(D_H2)(src_p, dst_p, xw2, zeros32)

  outp = pl.pallas_call(
      _tc_f_kernel,
      out_shape=jax.ShapeDtypeStruct((N_PAD, D_OUT), jnp.float32),
  )(acc2, xw2, degp, b2, Wfc, bfc)

  return outp[:N_NODES]


# speedup vs baseline: 28.9329x; 1.3704x over previous
import functools

import jax
import jax.numpy as jnp
from jax import lax
from jax.experimental import pallas as pl
from jax.experimental.pallas import tpu as pltpu
from jax.experimental.pallas import tpu_sc as plsc

N_NODES = 10000
N_EDGES = 160000
D_IN = 288
D_H1 = 16
D_H2 = 32
D_OUT = 7

NC = 2
NS = 16
NW = NC * NS

N_PAD = 10240
ROWS_PER_TILE = N_PAD // NS
PAD_DST = N_NODES + 8
PAD_SRC = N_PAD - 1

CHUNK = 128
E_PER_W = 5120
N_CHUNKS = E_PER_W // CHUNK
E_PAD = NW * E_PER_W


def _sc_mesh():
  return plsc.VectorSubcoreMesh(core_axis_name="c", subcore_axis_name="s")


GRP = 8
N_GRP = N_CHUNKS // GRP


def _deg_kernel(dst_hbm, ones_hbm, zeros_hbm, out_hbm,
                idx_v, ones_v, acc_sh, ssem):
  c = lax.axis_index("c")
  s = lax.axis_index("s")
  row0 = s * ROWS_PER_TILE

  pltpu.sync_copy(zeros_hbm.at[pl.ds(row0, ROWS_PER_TILE)],
                  acc_sh.at[pl.ds(row0, ROWS_PER_TILE)])
  pltpu.sync_copy(ones_hbm, ones_v)
  pltpu.sync_copy(dst_hbm.at[c, s], idx_v)
  plsc.subcore_barrier()

  def body(g, carry):
    jj = g * GRP
    hs = [
        pltpu.async_copy(ones_v, acc_sh.at[idx_v.at[jj + b]],
                         ssem.at[b], add=True)
        for b in range(GRP)
    ]
    for h in hs:
      h.wait()
    return carry

  lax.fori_loop(0, N_GRP, body, 0)
  plsc.subcore_barrier()
  pltpu.sync_copy(acc_sh.at[pl.ds(row0, ROWS_PER_TILE)],
                  out_hbm.at[c, pl.ds(row0, ROWS_PER_TILE)])


def _make_deg_call():
  return pl.kernel(
      _deg_kernel,
      out_type=jax.ShapeDtypeStruct((NC, N_PAD, 16), jnp.float32),
      mesh=_sc_mesh(),
      compiler_params=pltpu.CompilerParams(use_tc_tiling_on_sc=False),
      scratch_types=[
          pltpu.VMEM((N_CHUNKS, CHUNK), jnp.int32),
          pltpu.VMEM((CHUNK, 16), jnp.float32),
          pltpu.VMEM_SHARED((N_PAD, 16), jnp.float32),
          pltpu.SemaphoreType.DMA((GRP,)),
      ],
  )


def _edge_agg_kernel(src_hbm, dst_hbm, table_hbm, zeros_hbm, out_hbm,
                     src_v, dst_v, rows_v, acc_sh, table_sh, gsem, ssem):
  c = lax.axis_index("c")
  s = lax.axis_index("s")
  row0 = s * ROWS_PER_TILE

  pltpu.sync_copy(zeros_hbm.at[pl.ds(row0, ROWS_PER_TILE)],
                  acc_sh.at[pl.ds(row0, ROWS_PER_TILE)])
  pltpu.sync_copy(table_hbm.at[pl.ds(row0, ROWS_PER_TILE)],
                  table_sh.at[pl.ds(row0, ROWS_PER_TILE)])
  pltpu.sync_copy(src_hbm.at[c, s], src_v)
  pltpu.sync_copy(dst_hbm.at[c, s], dst_v)
  plsc.subcore_barrier()

  def body(g, carry):
    jj = g * GRP
    ghs = [
        pltpu.async_copy(table_sh.at[src_v.at[jj + b]], rows_v.at[b],
                         gsem.at[b])
        for b in range(GRP)
    ]
    shs = []
    for b in range(GRP):
      ghs[b].wait()
      shs.append(
          pltpu.async_copy(rows_v.at[b], acc_sh.at[dst_v.at[jj + b]],
                           ssem.at[b], add=True))
    for h in shs:
      h.wait()
    return carry

  lax.fori_loop(0, N_GRP, body, 0)
  plsc.subcore_barrier()
  pltpu.sync_copy(acc_sh.at[pl.ds(row0, ROWS_PER_TILE)],
                  out_hbm.at[c, pl.ds(row0, ROWS_PER_TILE)])


def _make_edge_agg_call(d):
  return pl.kernel(
      _edge_agg_kernel,
      out_type=jax.ShapeDtypeStruct((NC, N_PAD, d), jnp.float32),
      mesh=_sc_mesh(),
      compiler_params=pltpu.CompilerParams(use_tc_tiling_on_sc=False),
      scratch_types=[
          pltpu.VMEM((N_CHUNKS, CHUNK), jnp.int32),
          pltpu.VMEM((N_CHUNKS, CHUNK), jnp.int32),
          pltpu.VMEM((GRP, CHUNK, d), jnp.float32),
          pltpu.VMEM_SHARED((N_PAD, d), jnp.float32),
          pltpu.VMEM_SHARED((N_PAD, d), jnp.float32),
          pltpu.SemaphoreType.DMA((GRP,)),
          pltpu.SemaphoreType.DMA((GRP,)),
      ],
  )


def _dinv_from_degp(degp_ref):
  deg = degp_ref[0, :, 0] + degp_ref[1, :, 0] + 1.0
  return lax.rsqrt(deg)


def _tc_b_kernel(x_ref, w1_ref, degp_ref, xw_ref):
  dinv = _dinv_from_degp(degp_ref)
  xw = jnp.dot(x_ref[...], w1_ref[...], preferred_element_type=jnp.float32)
  xw_ref[:N_NODES] = xw * dinv[:N_NODES, None]
  xw_ref[N_NODES:] = jnp.zeros((N_PAD - N_NODES, D_H1), jnp.float32)


def _tc_d_kernel(acc_ref, xw1_ref, degp_ref, b1_ref, w2_ref, xw2_ref):
  dinv = _dinv_from_degp(degp_ref)
  h1 = dinv[:, None] * (acc_ref[0] + acc_ref[1] + xw1_ref[...]) + b1_ref[...]
  h1 = jnp.maximum(h1, 0.0)
  xw2 = jnp.dot(h1, w2_ref[...], preferred_element_type=jnp.float32)
  xw2_ref[...] = xw2 * dinv[:, None]


def _tc_f_kernel(acc_ref, xw2_ref, degp_ref, b2_ref, wfc_ref, bfc_ref, out_ref):
  dinv = _dinv_from_degp(degp_ref)
  h2 = dinv[:, None] * (acc_ref[0] + acc_ref[1] + xw2_ref[...]) + b2_ref[...]
  h2 = jnp.maximum(h2, 0.0)
  out_ref[...] = (
      jnp.dot(h2, wfc_ref[...], preferred_element_type=jnp.float32)
      + bfc_ref[...]
  )


@jax.jit
def kernel(x, edge_index, W1, b1, W2, b2, Wfc, bfc):
  src = edge_index[0].astype(jnp.int32)
  dst = edge_index[1].astype(jnp.int32)
  n_extra = E_PAD - N_EDGES
  src_p = jnp.concatenate(
      [src, jnp.full((n_extra,), PAD_SRC, jnp.int32)]
  ).reshape(NC, NS, N_CHUNKS, CHUNK)
  dst_p = jnp.concatenate(
      [dst, jnp.full((n_extra,), PAD_DST, jnp.int32)]
  ).reshape(NC, NS, N_CHUNKS, CHUNK)
  ones16 = jnp.ones((CHUNK, 16), jnp.float32)
  zeros16 = jnp.zeros((N_PAD, 16), jnp.float32)
  zeros32 = jnp.zeros((N_PAD, 32), jnp.float32)

  degp = _make_deg_call()(dst_p, ones16, zeros16)

  xw1 = pl.pallas_call(
      _tc_b_kernel,
      out_shape=jax.ShapeDtypeStruct((N_PAD, D_H1), jnp.float32),
  )(x, W1, degp)

  acc1 = _make_edge_agg_call(D_H1)(src_p, dst_p, xw1, zeros16)

  xw2 = pl.pallas_call(
      _tc_d_kernel,
      out_shape=jax.ShapeDtypeStruct((N_PAD, D_H2), jnp.float32),
  )(acc1, xw1, degp, b1, W2)

  acc2 = _make_edge_agg_call(D_H2)(src_p, dst_p, xw2, zeros32)

  outp = pl.pallas_call(
      _tc_f_kernel,
      out_shape=jax.ShapeDtypeStruct((N_PAD, D_OUT), jnp.float32),
  )(acc2, xw2, degp, b2, Wfc, bfc)

  return outp[:N_NODES]
